# Initial kernel scaffold; baseline (speedup 1.0000x reference)
import jax
import jax.numpy as jnp
from jax.experimental import pallas as pl


def kernel(x, edge_index, batch, global_features, emb, W1, b1, ln1_g, ln1_b, W2, b2, ln2_g, ln2_b, gW1, gb1, gW2, gb2, cW1, cb1, cW2, cb2):
    raise NotImplementedError("write your pallas kernel here")


# speedup vs baseline: 8.9906x; 8.9906x over previous
import functools

import jax
import jax.numpy as jnp
from jax import lax
from jax.experimental import pallas as pl
from jax.experimental.pallas import tpu as pltpu
from jax.experimental.pallas import tpu_sc as plsc

N = 10000
E = 320000
NG = 64
VOCAB = 400
TIN = 143
H1 = 256
H2 = 256
DC = 128
KCH = 80
NT = 16
EPT = E // NT
NCH = EPT // KCH
EPT32 = E // 32
NCH32 = EPT32 // KCH
BP = 2000
PH = 5120
ACC_R = PH + 8
OUT_R = 2 * PH
ZPT_A = ACC_R // 16
ZPT_B = ACC_R - 15 * ZPT_A
CPT = PH // 16


def _zero_acc(zeros_hbm, acc, sid):
    base = pl.multiple_of(sid * ZPT_A, 8)

    @pl.when(sid < 15)
    def _():
        pltpu.sync_copy(zeros_hbm.at[pl.ds(base, ZPT_A)],
                        acc.at[pl.ds(base, ZPT_A)])

    @pl.when(sid == 15)
    def _():
        pltpu.sync_copy(zeros_hbm.at[pl.ds(15 * ZPT_A, ZPT_B)],
                        acc.at[pl.ds(15 * ZPT_A, ZPT_B)])


def _copy_out(acc, out_hbm, cid, p, sid):
    base = pl.multiple_of(sid * CPT, 8)
    pltpu.sync_copy(acc.at[pl.ds(base, CPT)],
                    out_hbm.at[cid, pl.ds(p * PH + base, CPT)])


@functools.lru_cache(maxsize=None)
def _make_deg():
    mesh = plsc.VectorSubcoreMesh(core_axis_name="c", subcore_axis_name="s")

    @functools.partial(
        pl.kernel, mesh=mesh,
        out_type=jax.ShapeDtypeStruct((2, OUT_R, DC), jnp.float32),
        scratch_types=[
            pltpu.VMEM((NCH32, KCH), jnp.int32),
            pltpu.VMEM((KCH, DC), jnp.float32),
            pltpu.VMEM_SHARED((ACC_R, DC), jnp.float32),
        ],
    )
    def _deg_kernel(dst32p_hbm, ones_hbm, zeros_hbm, out_hbm, idx_d, ones_v, acc):
        cid = lax.axis_index("c")
        sid = lax.axis_index("s")
        wid = sid * 2 + cid
        pltpu.sync_copy(ones_hbm, ones_v)
        for p in range(2):
            _zero_acc(zeros_hbm, acc, sid)
            pltpu.sync_copy(dst32p_hbm.at[p, wid], idx_d)
            plsc.subcore_barrier()

            def body(j, carry):
                pltpu.sync_copy(ones_v, acc.at[idx_d.at[j]], add=True)
                return carry

            lax.fori_loop(0, NCH32, body, 0)
            plsc.subcore_barrier()
            _copy_out(acc, out_hbm, cid, p, sid)
            plsc.subcore_barrier()

    return _deg_kernel


@functools.lru_cache(maxsize=None)
def _make_agg():
    mesh = plsc.VectorSubcoreMesh(core_axis_name="c", subcore_axis_name="s")

    @functools.partial(
        pl.kernel, mesh=mesh,
        out_type=jax.ShapeDtypeStruct((2, OUT_R, DC), jnp.float32),
        scratch_types=[
            pltpu.VMEM((NCH + 2, KCH), jnp.int32),
            pltpu.VMEM((NCH, KCH), jnp.int32),
            pltpu.VMEM((KCH, DC), jnp.float32),
            pltpu.VMEM((KCH, DC), jnp.float32),
            pltpu.VMEM_SHARED((ACC_R, DC), jnp.float32),
            pltpu.SemaphoreType.DMA,
            pltpu.SemaphoreType.DMA,
        ],
    )
    def agg(h_hbm, src_hbm, dstp_hbm, zeros_hbm, out_hbm,
            idx_s, idx_d, rows0, rows1, acc, sem0, sem1):
        cid = lax.axis_index("c")
        sid = lax.axis_index("s")
        pltpu.sync_copy(src_hbm.at[cid, sid], idx_s)
        for p in range(2):
            _zero_acc(zeros_hbm, acc, sid)
            pltpu.sync_copy(dstp_hbm.at[p, sid], idx_d)
            plsc.subcore_barrier()

            pltpu.async_copy(h_hbm.at[idx_s.at[0]], rows0, sem0)

            def body(jj, carry):
                j0 = jj * 2
                j1 = j0 + 1
                j2 = j0 + 2
                pltpu.async_copy(h_hbm.at[idx_s.at[j1]], rows1, sem1)
                pltpu.make_async_copy(h_hbm.at[idx_s.at[j0]], rows0, sem0).wait()
                pltpu.sync_copy(rows0, acc.at[idx_d.at[j0]], add=True)
                pltpu.async_copy(h_hbm.at[idx_s.at[j2]], rows0, sem0)
                pltpu.make_async_copy(h_hbm.at[idx_s.at[j1]], rows1, sem1).wait()
                pltpu.sync_copy(rows1, acc.at[idx_d.at[j1]], add=True)
                return carry

            lax.fori_loop(0, NCH // 2, body, 0)
            pltpu.make_async_copy(h_hbm.at[idx_s.at[NCH]], rows0, sem0).wait()
            plsc.subcore_barrier()
            _copy_out(acc, out_hbm, cid, p, sid)
            plsc.subcore_barrier()

    return agg


def _deg_call(dst32p, ones_rows, z128):
    return _make_deg()(dst32p, ones_rows, z128)


def _agg_call(h_flat, srcp, dstp, z128):
    return _make_agg()(h_flat, srcp, dstp, z128)


def _prep_body(x_ref, emb_ref, deg_ref, w_ref, out_ref):
    xb = x_ref[...]
    t = jnp.clip(xb[:, 0].astype(jnp.int32), 0, VOCAB - 1)
    oh = (t[:, None] == lax.broadcasted_iota(jnp.int32, (BP, VOCAB), 1))
    embrow = jnp.dot(oh.astype(jnp.float32), emb_ref[...],
                     preferred_element_type=jnp.float32)
    deg = deg_ref[0, :, 0] + deg_ref[1, :, 0] + 1.0
    dinv = lax.rsqrt(deg)[:, None]
    h0 = jnp.concatenate([xb, embrow, jnp.zeros((BP, 16), jnp.float32)], axis=1)
    z = jnp.dot(h0, w_ref[...], preferred_element_type=jnp.float32) * dinv
    out_ref[0] = z[:, :DC]
    out_ref[1] = z[:, DC:]


def _prep_call(x, emb, deg2, W1p):
    return pl.pallas_call(
        _prep_body,
        grid=(N // BP,),
        in_specs=[
            pl.BlockSpec((BP, 128), lambda i: (i, 0)),
            pl.BlockSpec((VOCAB, 16), lambda i: (0, 0)),
            pl.BlockSpec((2, BP, DC), lambda i: (0, i, 0)),
            pl.BlockSpec((160, H1), lambda i: (0, 0)),
        ],
        out_specs=pl.BlockSpec((2, BP, DC), lambda i: (0, i, 0)),
        out_shape=jax.ShapeDtypeStruct((2, N, DC), jnp.float32),
    )(x, emb, deg2, W1p)


def _mid_body(agg_ref, z_ref, deg_ref, b_ref, g_ref, be_ref, w_ref, out_ref):
    deg = deg_ref[0, :, 0] + deg_ref[1, :, 0] + 1.0
    dinv = lax.rsqrt(deg)[:, None]
    h = jnp.concatenate([agg_ref[0] + z_ref[0], agg_ref[1] + z_ref[1]],
                        axis=1) * dinv + b_ref[...]
    mu = jnp.mean(h, axis=1, keepdims=True)
    var = jnp.mean((h - mu) * (h - mu), axis=1, keepdims=True)
    h = (h - mu) / jnp.sqrt(var + 1e-5) * g_ref[...] + be_ref[...]
    h = jnp.maximum(h, 0.0)
    z = jnp.dot(h, w_ref[...], preferred_element_type=jnp.float32) * dinv
    out_ref[0] = z[:, :DC]
    out_ref[1] = z[:, DC:]


def _mid_call(agg0, z0, deg2, b1, g1, be1, W2):
    return pl.pallas_call(
        _mid_body,
        grid=(N // BP,),
        in_specs=[
            pl.BlockSpec((2, BP, DC), lambda i: (0, i, 0)),
            pl.BlockSpec((2, BP, DC), lambda i: (0, i, 0)),
            pl.BlockSpec((2, BP, DC), lambda i: (0, i, 0)),
            pl.BlockSpec((1, H1), lambda i: (0, 0)),
            pl.BlockSpec((1, H1), lambda i: (0, 0)),
            pl.BlockSpec((1, H1), lambda i: (0, 0)),
            pl.BlockSpec((H1, H2), lambda i: (0, 0)),
        ],
        out_specs=pl.BlockSpec((2, BP, DC), lambda i: (0, i, 0)),
        out_shape=jax.ShapeDtypeStruct((2, N, DC), jnp.float32),
    )(agg0, z0, deg2, b1, g1, be1, W2)


def _h2_body(agg_ref, z_ref, deg_ref, b_ref, g_ref, be_ref,
             gw1_ref, gb1_ref, gw2_ref, gb2_ref, h2_ref, gate_ref):
    deg = deg_ref[0, :, 0] + deg_ref[1, :, 0] + 1.0
    dinv = lax.rsqrt(deg)[:, None]
    h = jnp.concatenate([agg_ref[0] + z_ref[0], agg_ref[1] + z_ref[1]],
                        axis=1) * dinv + b_ref[...]
    mu = jnp.mean(h, axis=1, keepdims=True)
    var = jnp.mean((h - mu) * (h - mu), axis=1, keepdims=True)
    h = (h - mu) / jnp.sqrt(var + 1e-5) * g_ref[...] + be_ref[...]
    h = jnp.maximum(h, 0.0)
    h2_ref[...] = h
    gmid = jnp.maximum(
        jnp.dot(h, gw1_ref[...], preferred_element_type=jnp.float32)
        + gb1_ref[...], 0.0)
    gate_ref[...] = (jnp.dot(gmid, gw2_ref[...],
                             preferred_element_type=jnp.float32) + gb2_ref[...])


def _h2_call(agg1, z1, deg2, b2, g2, be2, gW1, gb1, gW2, gb2):
    return pl.pallas_call(
        _h2_body,
        grid=(N // BP,),
        in_specs=[
            pl.BlockSpec((2, BP, DC), lambda i: (0, i, 0)),
            pl.BlockSpec((2, BP, DC), lambda i: (0, i, 0)),
            pl.BlockSpec((2, BP, DC), lambda i: (0, i, 0)),
            pl.BlockSpec((1, H2), lambda i: (0, 0)),
            pl.BlockSpec((1, H2), lambda i: (0, 0)),
            pl.BlockSpec((1, H2), lambda i: (0, 0)),
            pl.BlockSpec((H2, 128), lambda i: (0, 0)),
            pl.BlockSpec((1, 128), lambda i: (0, 0)),
            pl.BlockSpec((128, 1), lambda i: (0, 0)),
            pl.BlockSpec((1, 1), lambda i: (0, 0)),
        ],
        out_specs=[
            pl.BlockSpec((BP, H2), lambda i: (i, 0)),
            pl.BlockSpec((BP, 1), lambda i: (i, 0)),
        ],
        out_shape=[
            jax.ShapeDtypeStruct((N, H2), jnp.float32),
            jax.ShapeDtypeStruct((N, 1), jnp.float32),
        ],
    )(agg1, z1, deg2, b2, g2, be2, gW1, gb1, gW2, gb2)


def _pool_body(h2_ref, gate_ref, batch_ref, gf_ref, cw1_ref, cb1_ref,
               cw2_ref, cb2_ref, out_ref):
    h2 = h2_ref[...]
    gate = gate_ref[...][:, 0]
    b = batch_ref[...][0]
    pm = b[None, :] == lax.broadcasted_iota(jnp.int32, (NG, N), 0)
    pf = pm.astype(jnp.float32)
    m = jnp.max(jnp.where(pm, gate[None, :], -1e30), axis=1)
    mn = jnp.sum(pf * m[:, None], axis=0)
    e = jnp.exp(gate - mn)
    s = jnp.sum(pf * e[None, :], axis=1)
    sn = jnp.sum(pf * s[:, None], axis=0)
    alpha = e / (sn + 1e-16)
    pooled = jnp.dot(pf, h2 * alpha[:, None], preferred_element_type=jnp.float32)
    comb = jnp.concatenate([pooled, gf_ref[...]], axis=1)
    c1 = jnp.maximum(
        jnp.dot(comb, cw1_ref[...], preferred_element_type=jnp.float32)
        + cb1_ref[...], 0.0)
    out_ref[...] = (jnp.dot(c1, cw2_ref[...], preferred_element_type=jnp.float32)
                    + cb2_ref[...])


def _pool_call(h2, gate, batch2, gf, cW1, cb1, cW2, cb2):
    return pl.pallas_call(
        _pool_body,
        in_specs=[
            pl.BlockSpec((N, H2), lambda: (0, 0)),
            pl.BlockSpec((N, 1), lambda: (0, 0)),
            pl.BlockSpec((1, N), lambda: (0, 0)),
            pl.BlockSpec((NG, 16), lambda: (0, 0)),
            pl.BlockSpec((H2 + 16, 128), lambda: (0, 0)),
            pl.BlockSpec((1, 128), lambda: (0, 0)),
            pl.BlockSpec((128, 2), lambda: (0, 0)),
            pl.BlockSpec((1, 2), lambda: (0, 0)),
        ],
        out_specs=pl.BlockSpec((NG, 2), lambda: (0, 0)),
        out_shape=jax.ShapeDtypeStruct((NG, 2), jnp.float32),
    )(h2, gate, batch2, gf, cW1, cb1, cW2, cb2)


def kernel(x, edge_index, batch, global_features, emb, W1, b1, ln1_g, ln1_b,
           W2, b2, ln2_g, ln2_b, gW1, gb1, gW2, gb2, cW1, cb1, cW2, cb2):
    src = edge_index[0]
    dst = edge_index[1]
    dl0 = jnp.where(dst < PH, dst, PH)
    dl1 = jnp.where(dst >= PH, dst - PH, PH)
    dst32p = jnp.stack([dl0.reshape(32, NCH32, KCH),
                        dl1.reshape(32, NCH32, KCH)])
    dstp = jnp.stack([dl0.reshape(NT, NCH, KCH),
                      dl1.reshape(NT, NCH, KCH)])
    srcr = src.reshape(NT, NCH, KCH)
    srcp = jnp.stack([srcr, srcr + N])
    srcp = jnp.concatenate([srcp, srcp[:, :, :2]], axis=2)
    ones_rows = jnp.ones((KCH, DC), jnp.float32)
    z128 = jnp.zeros((ACC_R, DC), jnp.float32)

    deg2 = _deg_call(dst32p, ones_rows, z128)

    W1p = jnp.concatenate(
        [jnp.zeros((1, H1), jnp.float32), W1,
         jnp.zeros((160 - TIN - 1, H1), jnp.float32)], axis=0)

    z0 = _prep_call(x, emb, deg2, W1p)
    agg0 = _agg_call(z0.reshape(2 * N, DC), srcp, dstp, z128)
    z1 = _mid_call(agg0, z0, deg2, b1.reshape(1, -1),
                   ln1_g.reshape(1, -1), ln1_b.reshape(1, -1), W2)
    agg1 = _agg_call(z1.reshape(2 * N, DC), srcp, dstp, z128)
    h2, gate = _h2_call(agg1, z1, deg2, b2.reshape(1, -1),
                        ln2_g.reshape(1, -1), ln2_b.reshape(1, -1),
                        gW1, gb1.reshape(1, -1), gW2.reshape(-1, 1),
                        gb2.reshape(1, 1))
    return _pool_call(h2, gate, batch.reshape(1, N), global_features,
                      cW1, cb1.reshape(1, -1), cW2, cb2.reshape(1, 2))
